# baseline (speedup 1.0000x reference)
import dataclasses
import functools

import jax
import jax.numpy as jnp
from jax import lax
from jax.experimental import pallas as pl
from jax.experimental.pallas import tpu as pltpu
from jax.experimental.pallas import tpu_sc as plsc

_L = 16
_NW = 32

_GATHER_DNUMS = lax.GatherDimensionNumbers(
    offset_dims=(), collapsed_slice_dims=(0,), start_index_map=(0,)
)


def _bcast_lane(vec, c):
    return lax.gather(
        vec,
        jnp.full((_L, 1), c, jnp.int32),
        _GATHER_DNUMS,
        slice_sizes=(1,),
        mode=lax.GatherScatterMode.PROMISE_IN_BOUNDS,
    )


def _pooled_embed(chars, table_flat_i32, B, W, C, D, V):
    DW = D // 2
    NG = DW // _L
    b_per_w = B // _NW
    CB = 4
    n_chunks = b_per_w // CB
    mesh = plsc.VectorSubcoreMesh(core_axis_name="c", subcore_axis_name="s")
    cp = pltpu.CompilerParams()
    fields = pltpu.CompilerParams.__dataclass_fields__
    if "needs_layout_passes" in fields:
        cp = dataclasses.replace(cp, needs_layout_passes=False)
    if "use_tc_tiling_on_sc" in fields:
        cp = dataclasses.replace(cp, use_tc_tiling_on_sc=False)

    @functools.partial(
        pl.kernel,
        mesh=mesh,
        compiler_params=cp,
        out_type=jax.ShapeDtypeStruct((B, W, D), jnp.float32),
        scratch_types=[
            pltpu.VMEM((V * DW,), jnp.int32),
            pltpu.VMEM((CB, W, C), jnp.int32),
            pltpu.VMEM((CB, W, D), jnp.float32),
        ],
    )
    def k(chars_hbm, table_hbm, out_hbm, table_v, idx_v, out_v):
        wid = lax.axis_index("s") * 2 + lax.axis_index("c")
        base_b = wid * b_per_w
        pltpu.sync_copy(table_hbm, table_v)
        high_mask = jnp.full((_L,), -65536, jnp.int32)
        lane = lax.iota(jnp.int32, _L)
        views = [table_v.at[pl.ds(g * _L, V * DW - g * _L)] for g in range(NG)]

        @pl.loop(0, n_chunks)
        def _(chunk):
            b0 = base_b + chunk * CB
            pltpu.sync_copy(chars_hbm.at[pl.ds(b0, CB)], idx_v)

            @pl.loop(0, CB)
            def _(bb):
                @plsc.parallel_loop(0, W, unroll=2)
                def _(ww):
                    idxs = idx_v[bb, ww, pl.ds(0, C)]
                    offs = idxs << 5
                    accs = None
                    for c in range(C):
                        addr = _bcast_lane(offs, c) + lane
                        rows = [
                            plsc.bitcast(
                                plsc.load_gather(views[g], [addr]), jnp.bfloat16
                            )
                            for g in range(NG)
                        ]
                        if accs is None:
                            accs = rows
                        else:
                            accs = [jnp.maximum(a, r) for a, r in zip(accs, rows)]
                    for g in range(NG):
                        acc_i = plsc.bitcast(accs[g], jnp.int32)
                        lo = plsc.bitcast(acc_i << 16, jnp.float32)
                        hi = plsc.bitcast(acc_i & high_mask, jnp.float32)
                        out_v[bb, ww, pl.ds(2 * g * _L, _L)] = lo
                        out_v[bb, ww, pl.ds((2 * g + 1) * _L, _L)] = hi

            pltpu.sync_copy(out_v, out_hbm.at[pl.ds(b0, CB)])

    return k(chars, table_flat_i32)


def kernel(words, chars, embed_weight):
    B, W, C = chars.shape
    V, D = embed_weight.shape
    tb = embed_weight.astype(jnp.bfloat16).reshape(V, D // 32, 2, 16)
    tb = tb.transpose(0, 1, 3, 2).reshape(V, D // 2, 2)
    table_i32 = jax.lax.bitcast_convert_type(tb, jnp.int32).reshape(V * (D // 2))
    parts = [B // 4, 3 * B // 8, 3 * B // 8]
    chars_i = chars.astype(jnp.int32)
    outs = []
    b0 = 0
    for p in parts:
        outs.append(
            _pooled_embed(chars_i[b0:b0 + p], table_i32, p, W, C, D, V)
        )
        b0 += p
    return jnp.concatenate(outs, axis=0)


# speedup vs baseline: 1.0827x; 1.0827x over previous
import dataclasses
import functools

import jax
import jax.numpy as jnp
from jax import lax
from jax.experimental import pallas as pl
from jax.experimental.pallas import tpu as pltpu
from jax.experimental.pallas import tpu_sc as plsc

_L = 16
_NW = 32

_GATHER_DNUMS = lax.GatherDimensionNumbers(
    offset_dims=(), collapsed_slice_dims=(0,), start_index_map=(0,)
)


def _bcast_lane(vec, c):
    return lax.gather(
        vec,
        jnp.full((_L, 1), c, jnp.int32),
        _GATHER_DNUMS,
        slice_sizes=(1,),
        mode=lax.GatherScatterMode.PROMISE_IN_BOUNDS,
    )


def _pooled_embed(chars, table_flat_i32, B, W, C, D, V):
    DW = D // 2
    NG = DW // _L
    b_per_w = B // _NW
    CB = 4
    n_chunks = b_per_w // CB
    mesh = plsc.VectorSubcoreMesh(core_axis_name="c", subcore_axis_name="s")
    cp = pltpu.CompilerParams()
    fields = pltpu.CompilerParams.__dataclass_fields__
    if "needs_layout_passes" in fields:
        cp = dataclasses.replace(cp, needs_layout_passes=False)
    if "use_tc_tiling_on_sc" in fields:
        cp = dataclasses.replace(cp, use_tc_tiling_on_sc=False)

    @functools.partial(
        pl.kernel,
        mesh=mesh,
        compiler_params=cp,
        out_type=jax.ShapeDtypeStruct((B, W, D), jnp.float32),
        scratch_types=[
            pltpu.VMEM((V * DW,), jnp.int32),
            pltpu.VMEM((CB, W, C), jnp.int32),
            pltpu.VMEM((CB, W, D), jnp.float32),
        ],
    )
    def k(chars_hbm, table_hbm, out_hbm, table_v, idx_v, out_v):
        wid = lax.axis_index("s") * 2 + lax.axis_index("c")
        base_b = wid * b_per_w
        pltpu.sync_copy(table_hbm, table_v)
        high_mask = jnp.full((_L,), -65536, jnp.int32)
        lane = lax.iota(jnp.int32, _L)
        views = [table_v.at[pl.ds(g * _L, V * DW - g * _L)] for g in range(NG)]

        @pl.loop(0, n_chunks)
        def _(chunk):
            b0 = base_b + chunk * CB
            pltpu.sync_copy(chars_hbm.at[pl.ds(b0, CB)], idx_v)

            @pl.loop(0, CB)
            def _(bb):
                @plsc.parallel_loop(0, W, unroll=2)
                def _(ww):
                    idxs = idx_v[bb, ww, pl.ds(0, C)]
                    offs = idxs << 5
                    accs = None
                    for c in range(C):
                        addr = _bcast_lane(offs, c) + lane
                        rows = [
                            plsc.bitcast(
                                plsc.load_gather(views[g], [addr]), jnp.bfloat16
                            )
                            for g in range(NG)
                        ]
                        if accs is None:
                            accs = rows
                        else:
                            accs = [jnp.maximum(a, r) for a, r in zip(accs, rows)]
                    for g in range(NG):
                        acc_i = plsc.bitcast(accs[g], jnp.int32)
                        lo = plsc.bitcast(acc_i << 16, jnp.float32)
                        hi = plsc.bitcast(acc_i & high_mask, jnp.float32)
                        out_v[bb, ww, pl.ds(2 * g * _L, _L)] = lo
                        out_v[bb, ww, pl.ds((2 * g + 1) * _L, _L)] = hi

            pltpu.sync_copy(out_v, out_hbm.at[pl.ds(b0, CB)])

    return k(chars, table_flat_i32)


def kernel(words, chars, embed_weight):
    B, W, C = chars.shape
    V, D = embed_weight.shape
    tb = embed_weight.astype(jnp.bfloat16).reshape(V, D // 32, 2, 16)
    tb = tb.transpose(0, 1, 3, 2).reshape(V, D // 2, 2)
    table_i32 = jax.lax.bitcast_convert_type(tb, jnp.int32).reshape(V * (D // 2))
    parts = [3 * B // 8, 5 * B // 8]
    chars_i = chars.astype(jnp.int32)
    outs = []
    b0 = 0
    for p in parts:
        outs.append(
            _pooled_embed(chars_i[b0:b0 + p], table_i32, p, W, C, D, V)
        )
        b0 += p
    return jnp.concatenate(outs, axis=0)
